# baseline (speedup 1.0000x reference)
import functools

import jax
import jax.numpy as jnp
from jax.experimental import pallas as pl
from jax.experimental.pallas import tpu as pltpu

_M = 4096
_K = 512
_N = 65536
_BN = 1024
_BN_B = 2048


def _mm_kernel(a_ref, b_ref, o_ref):
    j = pl.program_id(1)
    a = a_ref[...].astype(jnp.bfloat16)
    b = b_ref[:, pl.ds(j * _BN, _BN)].astype(jnp.bfloat16)
    o_ref[...] = jnp.dot(a, b, preferred_element_type=jnp.float32)


@functools.partial(jax.jit, static_argnames=())
def kernel(query, queue):
    grid = (_N // _BN_B, _BN_B // _BN)
    return pl.pallas_call(
        _mm_kernel,
        grid=grid,
        in_specs=[
            pl.BlockSpec((_M, _K), lambda i, j: (0, 0)),
            pl.BlockSpec((_K, _BN_B), lambda i, j: (0, i)),
        ],
        out_specs=pl.BlockSpec((_M, _BN), lambda i, j: (0, i * (_BN_B // _BN) + j)),
        out_shape=jax.ShapeDtypeStruct((_M, _N), jnp.float32),
        compiler_params=pltpu.CompilerParams(
            dimension_semantics=("parallel", "arbitrary"),
            vmem_limit_bytes=63 * 1024 * 1024,
        ),
    )(query, queue)


# speedup vs baseline: 1.0160x; 1.0160x over previous
import functools

import jax
import jax.numpy as jnp
from jax.experimental import pallas as pl
from jax.experimental.pallas import tpu as pltpu

_M = 4096
_K = 512
_N = 65536
_BN = 1280


def _mm_kernel(a_ref, b_ref, o_ref):
    a = a_ref[...].astype(jnp.bfloat16)
    b = b_ref[...].astype(jnp.bfloat16)
    o_ref[...] = jnp.dot(a, b, preferred_element_type=jnp.float32)


@functools.partial(jax.jit, static_argnames=())
def kernel(query, queue):
    grid = (pl.cdiv(_N, _BN),)
    return pl.pallas_call(
        _mm_kernel,
        grid=grid,
        in_specs=[
            pl.BlockSpec((_M, _K), lambda j: (0, 0)),
            pl.BlockSpec((_K, _BN), lambda j: (0, j)),
        ],
        out_specs=pl.BlockSpec((_M, _BN), lambda j: (0, j)),
        out_shape=jax.ShapeDtypeStruct((_M, _N), jnp.float32),
        compiler_params=pltpu.CompilerParams(
            dimension_semantics=("arbitrary",),
            vmem_limit_bytes=63 * 1024 * 1024,
        ),
    )(query, queue)
